# baseline (speedup 1.0000x reference)
import functools

import jax
import jax.numpy as jnp
from jax import lax
from jax.experimental import pallas as pl
from jax.experimental.pallas import tpu as pltpu
from jax.experimental.pallas import tpu_sc as plsc

B = 16384
D = 64
NC = 2
NS = 16
NW = NC * NS
BW = B // NW
NG = BW // 16


def _transe_body(ent_hbm, rel_hbm, heads_hbm, rels_hbm, tails_hbm, out_hbm,
                 hvm, rvm, tvm, hrow, rrow, trow, outv, tmp, sem0, sem1):
    wid = lax.axis_index("s") * NC + lax.axis_index("c")
    base = wid * BW

    pltpu.sync_copy(heads_hbm.at[pl.ds(base, BW)], hvm)
    pltpu.sync_copy(rels_hbm.at[pl.ds(base, BW)], rvm)
    pltpu.sync_copy(tails_hbm.at[pl.ds(base, BW)], tvm)

    sems = (sem0, sem1)
    lanes = lax.iota(jnp.int32, 16)

    def issue_group(g, par):
        sem = sems[par]
        hvec = hvm[pl.ds(g * 16, 16)] * D
        rvec = rvm[pl.ds(g * 16, 16)] * D
        tvec = tvm[pl.ds(g * 16, 16)] * D
        for jj in range(16):
            rb = (par * 16 + jj) * D
            hb = pl.multiple_of(hvec[jj], D)
            pltpu.async_copy(ent_hbm.at[pl.ds(hb, D)],
                             hrow.at[pl.ds(rb, D)], sem)
            cb = pl.multiple_of(rvec[jj], D)
            pltpu.async_copy(rel_hbm.at[pl.ds(cb, D)],
                             rrow.at[pl.ds(rb, D)], sem)
            tb = pl.multiple_of(tvec[jj], D)
            pltpu.async_copy(ent_hbm.at[pl.ds(tb, D)],
                             trow.at[pl.ds(rb, D)], sem)

    def wait_group(par):
        sem = sems[par]
        for _ in range(48):
            pltpu.make_async_copy(ent_hbm.at[pl.ds(0, D)],
                                  hrow.at[pl.ds(0, D)], sem).wait()

    def compute_group(g, par):
        for jj in range(16):
            rb = (par * 16 + jj) * D
            acc = jnp.zeros((16,), jnp.float32)
            for k in range(D // 16):
                hv = hrow[pl.ds(rb + k * 16, 16)]
                rv = rrow[pl.ds(rb + k * 16, 16)]
                tv = trow[pl.ds(rb + k * 16, 16)]
                acc = acc + jnp.abs(hv + rv - tv)
            plsc.store_scatter(tmp, [lanes * 16 + jj], acc)
        colsum = jnp.zeros((16,), jnp.float32)
        for l in range(16):
            colsum = colsum + tmp[pl.ds(l * 16, 16)]
        outv[pl.ds(g * 16, 16)] = -colsum

    issue_group(0, 0)
    issue_group(1, 1)

    def step(gg, carry):
        for par in range(2):
            g = gg * 2 + par
            wait_group(par)
            compute_group(g, par)
            issue_group(g + 2, par)
        return carry

    lax.fori_loop(0, NG // 2 - 1, step, 0)
    for par in range(2):
        g = NG - 2 + par
        wait_group(par)
        compute_group(g, par)

    pltpu.sync_copy(outv, out_hbm.at[pl.ds(base, BW)])


def kernel(entity_table, relation_table, heads, relations, tails):
    ent_flat = entity_table.reshape(-1)
    rel_flat = relation_table.reshape(-1)
    mesh = plsc.VectorSubcoreMesh(core_axis_name="c", subcore_axis_name="s")
    run = functools.partial(
        pl.kernel,
        mesh=mesh,
        compiler_params=pltpu.CompilerParams(
            needs_layout_passes=False, use_tc_tiling_on_sc=True),
        out_type=jax.ShapeDtypeStruct((B,), jnp.float32),
        scratch_types=[
            pltpu.VMEM((BW,), jnp.int32),
            pltpu.VMEM((BW,), jnp.int32),
            pltpu.VMEM((BW,), jnp.int32),
            pltpu.VMEM((32 * D,), jnp.float32),
            pltpu.VMEM((32 * D,), jnp.float32),
            pltpu.VMEM((32 * D,), jnp.float32),
            pltpu.VMEM((BW,), jnp.float32),
            pltpu.VMEM((256,), jnp.float32),
            pltpu.SemaphoreType.DMA,
            pltpu.SemaphoreType.DMA,
        ],
    )(_transe_body)
    return run(ent_flat, rel_flat, heads, relations, tails)


# speedup vs baseline: 1.0030x; 1.0030x over previous
import functools

import jax
import jax.numpy as jnp
from jax import lax
from jax.experimental import pallas as pl
from jax.experimental.pallas import tpu as pltpu
from jax.experimental.pallas import tpu_sc as plsc

B = 16384
D = 64
NC = 2
NS = 16
NW = NC * NS
BW = B // NW
CH = 128
NCH = BW // CH


def _transe_body(ent_hbm, rel_hbm, heads_hbm, rels_hbm, tails_hbm, out_hbm,
                 hvm, rvm, tvm, hix, rix, tix, hbuf, rbuf, tbuf, outv, tmp,
                 sem0, sem1):
    wid = lax.axis_index("s") * NC + lax.axis_index("c")
    base = wid * BW

    pltpu.sync_copy(heads_hbm.at[pl.ds(base, BW)], hvm)
    pltpu.sync_copy(rels_hbm.at[pl.ds(base, BW)], rvm)
    pltpu.sync_copy(tails_hbm.at[pl.ds(base, BW)], tvm)
    for q in range(BW // 16):
        sl = pl.ds(q * 16, 16)
        hix[sl] = lax.shift_right_logical(hvm[sl], 1)
        rix[sl] = lax.shift_right_logical(rvm[sl], 1)
        tix[sl] = lax.shift_right_logical(tvm[sl], 1)

    sems = (sem0, sem1)
    lanes = lax.iota(jnp.int32, 16)

    def issue_chunk(c, par):
        sem = sems[par]
        sl = pl.ds(c * CH, CH)
        pltpu.async_copy(ent_hbm.at[hix.at[sl]], hbuf.at[par], sem)
        pltpu.async_copy(rel_hbm.at[rix.at[sl]], rbuf.at[par], sem)
        pltpu.async_copy(ent_hbm.at[tix.at[sl]], tbuf.at[par], sem)

    def wait_chunk(par):
        sem = sems[par]
        for _ in range(3):
            pltpu.make_async_copy(ent_hbm.at[pl.ds(0, CH), :],
                                  hbuf.at[0], sem).wait()

    def halfsel(par, buf, row, hv16):
        vals = []
        for k in range(D // 16):
            lo = buf[par, row, pl.ds(k * 16, 16)]
            hi = buf[par, row, pl.ds(D + k * 16, 16)]
            vals.append(jnp.where(hv16 == 0, lo, hi))
        return vals

    def compute_chunk(c, par):
        def group(g, carry):
            sl = pl.ds(c * CH + g * 16, 16)
            hh = jnp.bitwise_and(hvm[sl], 1)
            rh = jnp.bitwise_and(rvm[sl], 1)
            th = jnp.bitwise_and(tvm[sl], 1)
            for jj in range(16):
                row = g * 16 + jj
                hsel = halfsel(par, hbuf, row, hh[jj])
                rsel = halfsel(par, rbuf, row, rh[jj])
                tsel = halfsel(par, tbuf, row, th[jj])
                acc = jnp.zeros((16,), jnp.float32)
                for k in range(D // 16):
                    acc = acc + jnp.abs(hsel[k] + rsel[k] - tsel[k])
                plsc.store_scatter(tmp, [lanes * 16 + jj], acc)
            colsum = jnp.zeros((16,), jnp.float32)
            for l in range(16):
                colsum = colsum + tmp[pl.ds(l * 16, 16)]
            outv[pl.ds(c * CH + g * 16, 16)] = -colsum
            return carry

        lax.fori_loop(0, CH // 16, group, 0)

    issue_chunk(0, 0)
    issue_chunk(1, 1)
    for c in range(NCH):
        par = c % 2
        wait_chunk(par)
        compute_chunk(c, par)
        if c + 2 < NCH:
            issue_chunk(c + 2, par)

    pltpu.sync_copy(outv, out_hbm.at[pl.ds(base, BW)])


def kernel(entity_table, relation_table, heads, relations, tails):
    e2 = entity_table.reshape(-1, 2 * D)
    r2 = relation_table.reshape(-1, 2 * D)
    mesh = plsc.VectorSubcoreMesh(core_axis_name="c", subcore_axis_name="s")
    run = functools.partial(
        pl.kernel,
        mesh=mesh,
        compiler_params=pltpu.CompilerParams(
            needs_layout_passes=False, use_tc_tiling_on_sc=True),
        out_type=jax.ShapeDtypeStruct((B,), jnp.float32),
        scratch_types=[
            pltpu.VMEM((BW,), jnp.int32),
            pltpu.VMEM((BW,), jnp.int32),
            pltpu.VMEM((BW,), jnp.int32),
            pltpu.VMEM((BW,), jnp.int32),
            pltpu.VMEM((BW,), jnp.int32),
            pltpu.VMEM((BW,), jnp.int32),
            pltpu.VMEM((2, CH, 2 * D), jnp.float32),
            pltpu.VMEM((2, CH, 2 * D), jnp.float32),
            pltpu.VMEM((2, CH, 2 * D), jnp.float32),
            pltpu.VMEM((BW,), jnp.float32),
            pltpu.VMEM((256,), jnp.float32),
            pltpu.SemaphoreType.DMA,
            pltpu.SemaphoreType.DMA,
        ],
    )(_transe_body)
    return run(e2, r2, heads, relations, tails)


# speedup vs baseline: 1.9558x; 1.9499x over previous
import functools

import jax
import jax.numpy as jnp
from jax import lax
from jax.experimental import pallas as pl
from jax.experimental.pallas import tpu as pltpu
from jax.experimental.pallas import tpu_sc as plsc

B = 16384
D = 64
NC = 2
NS = 16
NW = NC * NS
BW = B // NW
NG = BW // 16


def _transe_body(ent_hbm, rel_hbm, heads_hbm, rels_hbm, tails_hbm, out_hbm,
                 hvm, rvm, tvm, sh, sr, st, outv, tmp, sem0, sem1):
    wid = lax.axis_index("s") * NC + lax.axis_index("c")
    base = wid * BW

    pltpu.sync_copy(heads_hbm.at[pl.ds(base, BW)], hvm)
    pltpu.sync_copy(rels_hbm.at[pl.ds(base, BW)], rvm)
    pltpu.sync_copy(tails_hbm.at[pl.ds(base, BW)], tvm)

    sems = (sem0, sem1)
    lanes = lax.iota(jnp.int32, 16)

    def issue_group(g, par):
        sem = sems[par]
        hvec = hvm[pl.ds(g * 16, 16)]
        rvec = rvm[pl.ds(g * 16, 16)]
        tvec = tvm[pl.ds(g * 16, 16)]
        for jj in range(16):
            rb = (par * 16 + jj) * 8
            hb = pl.multiple_of((hvec[jj] // 8) * 8, 8)
            pltpu.async_copy(ent_hbm.at[0, pl.ds(hb, 8), :],
                             sh.at[pl.ds(rb, 8), :], sem)
            cb = pl.multiple_of((rvec[jj] // 8) * 8, 8)
            pltpu.async_copy(rel_hbm.at[0, pl.ds(cb, 8), :],
                             sr.at[pl.ds(rb, 8), :], sem)
            tb = pl.multiple_of((tvec[jj] // 8) * 8, 8)
            pltpu.async_copy(ent_hbm.at[0, pl.ds(tb, 8), :],
                             st.at[pl.ds(rb, 8), :], sem)

    def wait_group(par):
        sem = sems[par]
        for _ in range(48):
            pltpu.make_async_copy(ent_hbm.at[0, pl.ds(0, 8), :],
                                  sh.at[pl.ds(0, 8), :], sem).wait()

    def compute_group(g, par):
        hvec = hvm[pl.ds(g * 16, 16)]
        rvec = rvm[pl.ds(g * 16, 16)]
        tvec = tvm[pl.ds(g * 16, 16)]
        for jj in range(16):
            rb = (par * 16 + jj) * 8
            hr = rb + lax.rem(hvec[jj], 8)
            rr = rb + lax.rem(rvec[jj], 8)
            tr = rb + lax.rem(tvec[jj], 8)
            acc = jnp.zeros((16,), jnp.float32)
            for k in range(D // 16):
                hv = sh[hr, pl.ds(k * 16, 16)]
                rv = sr[rr, pl.ds(k * 16, 16)]
                tv = st[tr, pl.ds(k * 16, 16)]
                acc = acc + jnp.abs(hv + rv - tv)
            plsc.store_scatter(tmp, [lanes * 16 + jj], acc)
        colsum = jnp.zeros((16,), jnp.float32)
        for l in range(16):
            colsum = colsum + tmp[pl.ds(l * 16, 16)]
        outv[pl.ds(g * 16, 16)] = -colsum

    issue_group(0, 0)
    issue_group(1, 1)

    def step(gg, carry):
        for par in range(2):
            g = gg * 2 + par
            wait_group(par)
            compute_group(g, par)
            issue_group(g + 2, par)
        return carry

    lax.fori_loop(0, NG // 2 - 1, step, 0)
    for par in range(2):
        g = NG - 2 + par
        wait_group(par)
        compute_group(g, par)

    pltpu.sync_copy(outv, out_hbm.at[pl.ds(base, BW)])


def kernel(entity_table, relation_table, heads, relations, tails):
    mesh = plsc.VectorSubcoreMesh(core_axis_name="c", subcore_axis_name="s")
    run = functools.partial(
        pl.kernel,
        mesh=mesh,
        compiler_params=pltpu.CompilerParams(
            needs_layout_passes=False, use_tc_tiling_on_sc=True),
        out_type=jax.ShapeDtypeStruct((B,), jnp.float32),
        scratch_types=[
            pltpu.VMEM((BW,), jnp.int32),
            pltpu.VMEM((BW,), jnp.int32),
            pltpu.VMEM((BW,), jnp.int32),
            pltpu.VMEM((256, D), jnp.float32),
            pltpu.VMEM((256, D), jnp.float32),
            pltpu.VMEM((256, D), jnp.float32),
            pltpu.VMEM((BW,), jnp.float32),
            pltpu.VMEM((256,), jnp.float32),
            pltpu.SemaphoreType.DMA,
            pltpu.SemaphoreType.DMA,
        ],
    )(_transe_body)
    e3 = entity_table.reshape(1, -1, D)
    r3 = relation_table.reshape(1, -1, D)
    return run(e3, r3, heads, relations, tails)


# speedup vs baseline: 1.9723x; 1.0084x over previous
import functools

import jax
import jax.numpy as jnp
from jax import lax
from jax.experimental import pallas as pl
from jax.experimental.pallas import tpu as pltpu
from jax.experimental.pallas import tpu_sc as plsc

B = 16384
D = 64
NC = 2
NS = 16
NW = NC * NS
BW = B // NW
NG = BW // 16


def _transe_body(ent_hbm, rel_hbm, heads_hbm, rels_hbm, tails_hbm, out_hbm,
                 hvm, rvm, tvm, sh, sr, st, outv, tmp, sem0, sem1):
    wid = lax.axis_index("s") * NC + lax.axis_index("c")
    base = wid * BW

    pltpu.sync_copy(heads_hbm.at[pl.ds(base, BW)], hvm)
    pltpu.sync_copy(rels_hbm.at[pl.ds(base, BW)], rvm)
    pltpu.sync_copy(tails_hbm.at[pl.ds(base, BW)], tvm)

    sems = (sem0, sem1)
    lanes = lax.iota(jnp.int32, 16)

    def issue_group(g, par):
        sem = sems[par]
        hvec = hvm[pl.ds(g * 16, 16)]
        rvec = rvm[pl.ds(g * 16, 16)]
        tvec = tvm[pl.ds(g * 16, 16)]
        for jj in range(16):
            rb = (par * 16 + jj) * 8
            hb = pl.multiple_of((hvec[jj] // 8) * 8, 8)
            pltpu.async_copy(ent_hbm.at[0, pl.ds(hb, 8), :],
                             sh.at[pl.ds(rb, 8), :], sem)
            cb = pl.multiple_of((rvec[jj] // 8) * 8, 8)
            pltpu.async_copy(rel_hbm.at[0, pl.ds(cb, 8), :],
                             sr.at[pl.ds(rb, 8), :], sem)
            tb = pl.multiple_of((tvec[jj] // 8) * 8, 8)
            pltpu.async_copy(ent_hbm.at[0, pl.ds(tb, 8), :],
                             st.at[pl.ds(rb, 8), :], sem)

    def wait_group(par):
        sem = sems[par]
        pltpu.make_async_copy(ent_hbm.at[0, pl.ds(0, 256), :],
                              sh, sem).wait()
        pltpu.make_async_copy(ent_hbm.at[0, pl.ds(0, 128), :],
                              sh.at[pl.ds(0, 128), :], sem).wait()

    def compute_group(g, par):
        hvec = hvm[pl.ds(g * 16, 16)]
        rvec = rvm[pl.ds(g * 16, 16)]
        tvec = tvm[pl.ds(g * 16, 16)]
        for jj in range(16):
            rb = (par * 16 + jj) * 8
            hr = rb + lax.rem(hvec[jj], 8)
            rr = rb + lax.rem(rvec[jj], 8)
            tr = rb + lax.rem(tvec[jj], 8)
            acc = jnp.zeros((16,), jnp.float32)
            for k in range(D // 16):
                hv = sh[hr, pl.ds(k * 16, 16)]
                rv = sr[rr, pl.ds(k * 16, 16)]
                tv = st[tr, pl.ds(k * 16, 16)]
                acc = acc + jnp.abs(hv + rv - tv)
            plsc.store_scatter(tmp, [lanes * 16 + jj], acc)
        colsum = jnp.zeros((16,), jnp.float32)
        for l in range(16):
            colsum = colsum + tmp[pl.ds(l * 16, 16)]
        outv[pl.ds(g * 16, 16)] = -colsum

    issue_group(0, 0)
    issue_group(1, 1)

    def step(gg, carry):
        for par in range(2):
            g = gg * 2 + par
            wait_group(par)
            compute_group(g, par)
            issue_group(g + 2, par)
        return carry

    lax.fori_loop(0, NG // 2 - 1, step, 0)
    for par in range(2):
        g = NG - 2 + par
        wait_group(par)
        compute_group(g, par)

    pltpu.sync_copy(outv, out_hbm.at[pl.ds(base, BW)])


def kernel(entity_table, relation_table, heads, relations, tails):
    mesh = plsc.VectorSubcoreMesh(core_axis_name="c", subcore_axis_name="s")
    run = functools.partial(
        pl.kernel,
        mesh=mesh,
        compiler_params=pltpu.CompilerParams(
            needs_layout_passes=False, use_tc_tiling_on_sc=True),
        out_type=jax.ShapeDtypeStruct((B,), jnp.float32),
        scratch_types=[
            pltpu.VMEM((BW,), jnp.int32),
            pltpu.VMEM((BW,), jnp.int32),
            pltpu.VMEM((BW,), jnp.int32),
            pltpu.VMEM((256, D), jnp.float32),
            pltpu.VMEM((256, D), jnp.float32),
            pltpu.VMEM((256, D), jnp.float32),
            pltpu.VMEM((BW,), jnp.float32),
            pltpu.VMEM((256,), jnp.float32),
            pltpu.SemaphoreType.DMA,
            pltpu.SemaphoreType.DMA,
        ],
    )(_transe_body)
    e3 = entity_table.reshape(1, -1, D)
    r3 = relation_table.reshape(1, -1, D)
    return run(e3, r3, heads, relations, tails)
